# Initial kernel scaffold; baseline (speedup 1.0000x reference)
import jax
import jax.numpy as jnp
from jax.experimental import pallas as pl


def kernel(x, edge_index, batch, W1, b1, W2, b2, Wlin, blin):
    raise NotImplementedError("write your pallas kernel here")


# speedup vs baseline: 5.9496x; 5.9496x over previous
import functools

import jax
import jax.numpy as jnp
from jax import lax
from jax.experimental import pallas as pl
from jax.experimental.pallas import tpu as pltpu
from jax.experimental.pallas import tpu_sc as plsc

N = 10000
E = 160000
G = 16
D_IN = 256
D_H = 128

N_PAD = 10240
E_PAD = 163840
CHUNK = 128
NC = 2
NS = 16
ROWS_PER_SUB = N_PAD // NS
CH_AGG = E_PAD // (NC * NS) // CHUNK
BLK = 256
NBLK = N_PAD // BLK


def _sc_mesh():
    return plsc.VectorSubcoreMesh(core_axis_name="c", subcore_axis_name="s")


def _deg_kernel(dst_hbm, ones_hbm, zeros_hbm, out_hbm,
                acc_sh, ones_v, dst_v, sem):
    cid = lax.axis_index("c")
    sid = lax.axis_index("s")
    pltpu.sync_copy(zeros_hbm, acc_sh.at[pl.ds(sid * ROWS_PER_SUB, ROWS_PER_SUB)])
    pltpu.sync_copy(ones_hbm, ones_v)
    plsc.subcore_barrier()
    base = (cid * NS + sid) * (CH_AGG * CHUNK)

    def body(g, carry):
        off = base + g * CHUNK
        pltpu.sync_copy(dst_hbm.at[pl.ds(off, CHUNK)], dst_v)
        pltpu.sync_copy(ones_v, acc_sh.at[dst_v], add=True)
        return carry

    lax.fori_loop(0, CH_AGG, body, 0)
    plsc.subcore_barrier()
    pltpu.sync_copy(acc_sh.at[pl.ds(sid * ROWS_PER_SUB, ROWS_PER_SUB)],
                    out_hbm.at[cid, pl.ds(sid * ROWS_PER_SUB, ROWS_PER_SUB)])


def _agg_kernel(z_hbm, src_hbm, dst_hbm, zeros_hbm, out_hbm,
                acc_sh, rows_v, idx_v, dst_v, sem):
    cid = lax.axis_index("c")
    sid = lax.axis_index("s")
    pltpu.sync_copy(zeros_hbm, acc_sh.at[pl.ds(sid * ROWS_PER_SUB, ROWS_PER_SUB)])
    plsc.subcore_barrier()
    base = (cid * NS + sid) * (CH_AGG * CHUNK)

    def body(g, carry):
        off = base + g * CHUNK
        pltpu.sync_copy(src_hbm.at[pl.ds(off, CHUNK)], idx_v)
        pltpu.sync_copy(dst_hbm.at[pl.ds(off, CHUNK)], dst_v)
        pltpu.async_copy(z_hbm.at[idx_v], rows_v, sem).wait()
        pltpu.sync_copy(rows_v, acc_sh.at[dst_v], add=True)
        return carry

    lax.fori_loop(0, CH_AGG, body, 0)
    plsc.subcore_barrier()
    pltpu.sync_copy(acc_sh.at[pl.ds(sid * ROWS_PER_SUB, ROWS_PER_SUB)],
                    out_hbm.at[cid, pl.ds(sid * ROWS_PER_SUB, ROWS_PER_SUB)])


def _make_deg():
    return pl.kernel(
        _deg_kernel,
        out_type=jax.ShapeDtypeStruct((NC, N_PAD, D_H), jnp.float32),
        mesh=_sc_mesh(),
        scratch_types=[
            pltpu.VMEM_SHARED((N_PAD, D_H), jnp.float32),
            pltpu.VMEM((CHUNK, D_H), jnp.float32),
            pltpu.VMEM((CHUNK,), jnp.int32),
            pltpu.SemaphoreType.DMA,
        ],
    )


def _make_agg():
    return pl.kernel(
        _agg_kernel,
        out_type=jax.ShapeDtypeStruct((NC, N_PAD, D_H), jnp.float32),
        mesh=_sc_mesh(),
        scratch_types=[
            pltpu.VMEM_SHARED((N_PAD, D_H), jnp.float32),
            pltpu.VMEM((CHUNK, D_H), jnp.float32),
            pltpu.VMEM((CHUNK,), jnp.int32),
            pltpu.VMEM((CHUNK,), jnp.int32),
            pltpu.SemaphoreType.DMA,
        ],
    )


def _k1_body(x_ref, w1_ref, degp_ref, z_ref):
    dinv = lax.rsqrt(degp_ref[0] + degp_ref[1] + 1.0)
    xw = jnp.dot(x_ref[...], w1_ref[...], preferred_element_type=jnp.float32)
    z_ref[...] = xw * dinv


def _k1(x_pad, W1, degp):
    return pl.pallas_call(
        _k1_body,
        grid=(NBLK,),
        in_specs=[
            pl.BlockSpec((BLK, D_IN), lambda i: (i, 0)),
            pl.BlockSpec((D_IN, D_H), lambda i: (0, 0)),
            pl.BlockSpec((NC, BLK, D_H), lambda i: (0, i, 0)),
        ],
        out_specs=pl.BlockSpec((BLK, D_H), lambda i: (i, 0)),
        out_shape=jax.ShapeDtypeStruct((N_PAD, D_H), jnp.float32),
    )(x_pad, W1, degp)


def _k2_body(s_ref, z1_ref, degp_ref, b1_ref, w2_ref, out_ref):
    dinv = lax.rsqrt(degp_ref[0] + degp_ref[1] + 1.0)
    h = jnp.maximum(dinv * (s_ref[0] + s_ref[1] + z1_ref[...]) + b1_ref[...], 0.0)
    out_ref[...] = jnp.dot(h, w2_ref[...], preferred_element_type=jnp.float32) * dinv


def _k2(S1, Z1, degp, b1r, W2):
    return pl.pallas_call(
        _k2_body,
        grid=(NBLK,),
        in_specs=[
            pl.BlockSpec((NC, BLK, D_H), lambda i: (0, i, 0)),
            pl.BlockSpec((BLK, D_H), lambda i: (i, 0)),
            pl.BlockSpec((NC, BLK, D_H), lambda i: (0, i, 0)),
            pl.BlockSpec((1, D_H), lambda i: (0, 0)),
            pl.BlockSpec((D_H, D_H), lambda i: (0, 0)),
        ],
        out_specs=pl.BlockSpec((BLK, D_H), lambda i: (i, 0)),
        out_shape=jax.ShapeDtypeStruct((N_PAD, D_H), jnp.float32),
    )(S1, Z1, degp, b1r, W2)


def _k3_body(s_ref, z2_ref, degp_ref, b2_ref, batch_ref, wlin_ref, blin_ref,
             out_ref, sums_ref, cnt_ref):
    i = pl.program_id(0)

    @pl.when(i == 0)
    def _():
        sums_ref[...] = jnp.zeros_like(sums_ref)
        cnt_ref[...] = jnp.zeros_like(cnt_ref)

    dinv = lax.rsqrt(degp_ref[0] + degp_ref[1] + 1.0)
    h2 = jnp.maximum(dinv * (s_ref[0] + s_ref[1] + z2_ref[...]) + b2_ref[...], 0.0)
    b = batch_ref[0]
    gids = lax.broadcasted_iota(jnp.int32, (G, BLK), 0)
    oh = (gids == b).astype(jnp.float32)
    sums_ref[...] += jnp.dot(oh, h2, preferred_element_type=jnp.float32)
    cnt_ref[...] += jnp.broadcast_to(
        jnp.sum(oh, axis=1, keepdims=True), (G, D_H))

    @pl.when(i == NBLK - 1)
    def _():
        pooled = sums_ref[...] / jnp.maximum(cnt_ref[...], 1.0)
        out_ref[...] = (
            jnp.dot(pooled, wlin_ref[...], preferred_element_type=jnp.float32)
            + blin_ref[...])


def _k3(S2, Z2, degp, b2r, batch3, wlin_pad, blin_pad):
    return pl.pallas_call(
        _k3_body,
        grid=(NBLK,),
        in_specs=[
            pl.BlockSpec((NC, BLK, D_H), lambda i: (0, i, 0)),
            pl.BlockSpec((BLK, D_H), lambda i: (i, 0)),
            pl.BlockSpec((NC, BLK, D_H), lambda i: (0, i, 0)),
            pl.BlockSpec((1, D_H), lambda i: (0, 0)),
            pl.BlockSpec((1, 1, BLK), lambda i: (i, 0, 0)),
            pl.BlockSpec((D_H, D_H), lambda i: (0, 0)),
            pl.BlockSpec((1, D_H), lambda i: (0, 0)),
        ],
        out_specs=pl.BlockSpec((G, D_H), lambda i: (0, 0)),
        out_shape=jax.ShapeDtypeStruct((G, D_H), jnp.float32),
        scratch_shapes=[
            pltpu.VMEM((G, D_H), jnp.float32),
            pltpu.VMEM((G, D_H), jnp.float32),
        ],
    )(S2, Z2, degp, b2r, batch3, wlin_pad, blin_pad)


@jax.jit
def kernel(x, edge_index, batch, W1, b1, W2, b2, Wlin, blin):
    src = edge_index[0]
    dst = edge_index[1]
    src_pad = jnp.full((E_PAD,), N, jnp.int32).at[:E].set(src)
    dst_pad = jnp.full((E_PAD,), N, jnp.int32).at[:E].set(dst)
    x_pad = jnp.zeros((N_PAD, D_IN), jnp.float32).at[:N, :].set(x)
    batch3 = jnp.full((N_PAD,), G, jnp.int32).at[:N].set(batch).reshape(NBLK, 1, BLK)
    ones_rows = jnp.ones((CHUNK, D_H), jnp.float32)
    zeros_rows = jnp.zeros((ROWS_PER_SUB, D_H), jnp.float32)
    b1r = b1.reshape(1, D_H)
    b2r = b2.reshape(1, D_H)
    wlin_pad = jnp.zeros((D_H, D_H), jnp.float32).at[:, :1].set(Wlin)
    blin_pad = jnp.zeros((1, D_H), jnp.float32).at[0, 0].set(blin[0])

    degp = _make_deg()(dst_pad, ones_rows, zeros_rows)
    Z1 = _k1(x_pad, W1, degp)
    S1 = _make_agg()(Z1, src_pad, dst_pad, zeros_rows)
    Z2 = _k2(S1, Z1, degp, b1r, W2)
    S2 = _make_agg()(Z2, src_pad, dst_pad, zeros_rows)
    outg = _k3(S2, Z2, degp, b2r, batch3, wlin_pad, blin_pad)
    return outg[:, 0].reshape(-1)


# speedup vs baseline: 7.3691x; 1.2386x over previous
import jax
import jax.numpy as jnp
from jax import lax
from jax.experimental import pallas as pl
from jax.experimental.pallas import tpu as pltpu
from jax.experimental.pallas import tpu_sc as plsc

N = 10000
E = 160000
G = 16
D_IN = 256
D_H = 128
DEG_W = 16

N_PAD = 10240
E_PAD = 163840
CHUNK = 128
NC = 2
NS = 16
NW = NC * NS
ROWS_PER_SUB = N_PAD // NS
CH_AGG = E_PAD // NW // CHUNK
NBUF = 2
BLK = 256
NBLK = N_PAD // BLK


def _sc_mesh():
    return plsc.VectorSubcoreMesh(core_axis_name="c", subcore_axis_name="s")


def _deg_kernel(dst_hbm, ones_hbm, zeros_hbm, out_hbm,
                acc_sh, ones_v, dstv, sem):
    cid = lax.axis_index("c")
    sid = lax.axis_index("s")
    wid = cid * NS + sid
    pltpu.sync_copy(zeros_hbm, acc_sh.at[pl.ds(sid * ROWS_PER_SUB, ROWS_PER_SUB)])
    pltpu.sync_copy(ones_hbm, ones_v)
    pltpu.sync_copy(dst_hbm.at[wid], dstv)
    plsc.subcore_barrier()

    def body(g, carry):
        pltpu.sync_copy(ones_v, acc_sh.at[dstv.at[g]], add=True)
        return carry

    lax.fori_loop(0, CH_AGG, body, 0)
    plsc.subcore_barrier()
    pltpu.sync_copy(acc_sh.at[pl.ds(sid * ROWS_PER_SUB, ROWS_PER_SUB)],
                    out_hbm.at[cid, pl.ds(sid * ROWS_PER_SUB, ROWS_PER_SUB)])


def _agg_kernel(z_hbm, src_hbm, dst_hbm, zeros_hbm, out_hbm,
                acc_sh, rows0, rows1, srcv, dstv, sem0, sem1):
    cid = lax.axis_index("c")
    sid = lax.axis_index("s")
    wid = cid * NS + sid
    rows = (rows0, rows1)
    sems = (sem0, sem1)
    pltpu.sync_copy(zeros_hbm, acc_sh.at[pl.ds(sid * ROWS_PER_SUB, ROWS_PER_SUB)])
    pltpu.sync_copy(src_hbm.at[wid], srcv)
    pltpu.sync_copy(dst_hbm.at[wid], dstv)
    plsc.subcore_barrier()

    for b in range(NBUF):
        pltpu.async_copy(z_hbm.at[srcv.at[b]], rows[b], sems[b])

    def body(h, carry):
        for b in range(NBUF):
            g = h * NBUF + b
            pltpu.make_async_copy(z_hbm.at[srcv.at[0]], rows[b], sems[b]).wait()
            pltpu.sync_copy(rows[b], acc_sh.at[dstv.at[g]], add=True)

            @pl.when(g + NBUF < CH_AGG)
            def _():
                pltpu.async_copy(z_hbm.at[srcv.at[g + NBUF]], rows[b], sems[b])
        return carry

    lax.fori_loop(0, CH_AGG // NBUF, body, 0)
    plsc.subcore_barrier()
    pltpu.sync_copy(acc_sh.at[pl.ds(sid * ROWS_PER_SUB, ROWS_PER_SUB)],
                    out_hbm.at[cid, pl.ds(sid * ROWS_PER_SUB, ROWS_PER_SUB)])


def _make_deg():
    return pl.kernel(
        _deg_kernel,
        out_type=jax.ShapeDtypeStruct((NC, N_PAD, DEG_W), jnp.float32),
        mesh=_sc_mesh(),
        scratch_types=[
            pltpu.VMEM_SHARED((N_PAD, DEG_W), jnp.float32),
            pltpu.VMEM((CHUNK, DEG_W), jnp.float32),
            pltpu.VMEM((CH_AGG, CHUNK), jnp.int32),
            pltpu.SemaphoreType.DMA,
        ],
    )


def _make_agg():
    return pl.kernel(
        _agg_kernel,
        out_type=jax.ShapeDtypeStruct((NC, N_PAD, D_H), jnp.float32),
        mesh=_sc_mesh(),
        scratch_types=[
            pltpu.VMEM_SHARED((N_PAD, D_H), jnp.float32),
            pltpu.VMEM((CHUNK, D_H), jnp.float32),
            pltpu.VMEM((CHUNK, D_H), jnp.float32),
            pltpu.VMEM((CH_AGG, CHUNK), jnp.int32),
            pltpu.VMEM((CH_AGG, CHUNK), jnp.int32),
            pltpu.SemaphoreType.DMA,
            pltpu.SemaphoreType.DMA,
        ],
    )


def _dinv_of(degp_ref):
    d16 = lax.rsqrt(degp_ref[0] + degp_ref[1] + 1.0)
    return jnp.broadcast_to(d16[:, :1], (d16.shape[0], D_H))


def _k1_body(x_ref, w1_ref, degp_ref, z_ref):
    xw = jnp.dot(x_ref[...], w1_ref[...], preferred_element_type=jnp.float32)
    z_ref[...] = xw * _dinv_of(degp_ref)


def _k1(x_pad, W1, degp):
    return pl.pallas_call(
        _k1_body,
        grid=(NBLK,),
        in_specs=[
            pl.BlockSpec((BLK, D_IN), lambda i: (i, 0)),
            pl.BlockSpec((D_IN, D_H), lambda i: (0, 0)),
            pl.BlockSpec((NC, BLK, DEG_W), lambda i: (0, i, 0)),
        ],
        out_specs=pl.BlockSpec((BLK, D_H), lambda i: (i, 0)),
        out_shape=jax.ShapeDtypeStruct((N_PAD, D_H), jnp.float32),
    )(x_pad, W1, degp)


def _k2_body(s_ref, z1_ref, degp_ref, b1_ref, w2_ref, out_ref):
    dinv = _dinv_of(degp_ref)
    h = jnp.maximum(dinv * (s_ref[0] + s_ref[1] + z1_ref[...]) + b1_ref[...], 0.0)
    out_ref[...] = jnp.dot(h * dinv, w2_ref[...], preferred_element_type=jnp.float32)


def _k2(S1, Z1, degp, b1r, W2):
    return pl.pallas_call(
        _k2_body,
        grid=(NBLK,),
        in_specs=[
            pl.BlockSpec((NC, BLK, D_H), lambda i: (0, i, 0)),
            pl.BlockSpec((BLK, D_H), lambda i: (i, 0)),
            pl.BlockSpec((NC, BLK, DEG_W), lambda i: (0, i, 0)),
            pl.BlockSpec((1, D_H), lambda i: (0, 0)),
            pl.BlockSpec((D_H, D_H), lambda i: (0, 0)),
        ],
        out_specs=pl.BlockSpec((BLK, D_H), lambda i: (i, 0)),
        out_shape=jax.ShapeDtypeStruct((N_PAD, D_H), jnp.float32),
    )(S1, Z1, degp, b1r, W2)


def _k3_body(s_ref, z2_ref, degp_ref, b2_ref, batch_ref, wlin_ref, blin_ref,
             out_ref, sums_ref, cnt_ref):
    i = pl.program_id(0)

    @pl.when(i == 0)
    def _():
        sums_ref[...] = jnp.zeros_like(sums_ref)
        cnt_ref[...] = jnp.zeros_like(cnt_ref)

    dinv = _dinv_of(degp_ref)
    h2 = jnp.maximum(dinv * (s_ref[0] + s_ref[1] + z2_ref[...]) + b2_ref[...], 0.0)
    b = batch_ref[0]
    gids = lax.broadcasted_iota(jnp.int32, (G, BLK), 0)
    oh = (gids == b).astype(jnp.float32)
    sums_ref[...] += jnp.dot(oh, h2, preferred_element_type=jnp.float32)
    cnt_ref[...] += jnp.broadcast_to(
        jnp.sum(oh, axis=1, keepdims=True), (G, D_H))

    @pl.when(i == NBLK - 1)
    def _():
        pooled = sums_ref[...] / jnp.maximum(cnt_ref[...], 1.0)
        out_ref[...] = (
            jnp.dot(pooled, wlin_ref[...], preferred_element_type=jnp.float32)
            + blin_ref[...])


def _k3(S2, Z2, degp, b2r, batch3, wlin_pad, blin_pad):
    return pl.pallas_call(
        _k3_body,
        grid=(NBLK,),
        in_specs=[
            pl.BlockSpec((NC, BLK, D_H), lambda i: (0, i, 0)),
            pl.BlockSpec((BLK, D_H), lambda i: (i, 0)),
            pl.BlockSpec((NC, BLK, DEG_W), lambda i: (0, i, 0)),
            pl.BlockSpec((1, D_H), lambda i: (0, 0)),
            pl.BlockSpec((1, 1, BLK), lambda i: (i, 0, 0)),
            pl.BlockSpec((D_H, D_H), lambda i: (0, 0)),
            pl.BlockSpec((1, D_H), lambda i: (0, 0)),
        ],
        out_specs=pl.BlockSpec((G, D_H), lambda i: (0, 0)),
        out_shape=jax.ShapeDtypeStruct((G, D_H), jnp.float32),
        scratch_shapes=[
            pltpu.VMEM((G, D_H), jnp.float32),
            pltpu.VMEM((G, D_H), jnp.float32),
        ],
    )(S2, Z2, degp, b2r, batch3, wlin_pad, blin_pad)


@jax.jit
def kernel(x, edge_index, batch, W1, b1, W2, b2, Wlin, blin):
    src = edge_index[0]
    dst = edge_index[1]
    src_pad = jnp.full((E_PAD,), N, jnp.int32).at[:E].set(src)
    dst_pad = jnp.full((E_PAD,), N, jnp.int32).at[:E].set(dst)
    src3 = src_pad.reshape(NW, CH_AGG, CHUNK)
    dst3 = dst_pad.reshape(NW, CH_AGG, CHUNK)
    x_pad = jnp.zeros((N_PAD, D_IN), jnp.float32).at[:N, :].set(x)
    batch3 = jnp.full((N_PAD,), G, jnp.int32).at[:N].set(batch).reshape(NBLK, 1, BLK)
    ones_rows = jnp.ones((CHUNK, DEG_W), jnp.float32)
    zeros_deg = jnp.zeros((ROWS_PER_SUB, DEG_W), jnp.float32)
    zeros_rows = jnp.zeros((ROWS_PER_SUB, D_H), jnp.float32)
    b1r = b1.reshape(1, D_H)
    b2r = b2.reshape(1, D_H)
    wlin_pad = jnp.zeros((D_H, D_H), jnp.float32).at[:, :1].set(Wlin)
    blin_pad = jnp.zeros((1, D_H), jnp.float32).at[0, 0].set(blin[0])

    degp = _make_deg()(dst3, ones_rows, zeros_deg)
    Z1 = _k1(x_pad, W1, degp)
    S1 = _make_agg()(Z1, src3, dst3, zeros_rows)
    Z2 = _k2(S1, Z1, degp, b1r, W2)
    S2 = _make_agg()(Z2, src3, dst3, zeros_rows)
    outg = _k3(S2, Z2, degp, b2r, batch3, wlin_pad, blin_pad)
    return outg[:, 0].reshape(-1)


# speedup vs baseline: 17.7567x; 2.4096x over previous
import jax
import jax.numpy as jnp
from jax import lax
from jax.experimental import pallas as pl
from jax.experimental.pallas import tpu as pltpu
from jax.experimental.pallas import tpu_sc as plsc

N = 10000
E = 160000
G = 16
D_IN = 256
D_H = 128
DEG_W = 16

N_PAD = 10240
E_PAD = 163840
CHUNK = 128
NC = 2
NS = 16
NW = NC * NS
ROWS_PER_SUB = N_PAD // NS
CH_AGG = E_PAD // NW // CHUNK
NBUF = 2
BLK = 256
NBLK = N_PAD // BLK


def _sc_mesh():
    return plsc.VectorSubcoreMesh(core_axis_name="c", subcore_axis_name="s")


def _deg_kernel(dst_hbm, ones_hbm, zeros_hbm, out_hbm,
                acc_sh, ones_v, dstv, sem):
    cid = lax.axis_index("c")
    sid = lax.axis_index("s")
    wid = cid * NS + sid
    pltpu.sync_copy(zeros_hbm, acc_sh.at[pl.ds(sid * ROWS_PER_SUB, ROWS_PER_SUB)])
    pltpu.sync_copy(ones_hbm, ones_v)
    pltpu.sync_copy(dst_hbm.at[wid], dstv)
    plsc.subcore_barrier()

    def body(g, carry):
        pltpu.sync_copy(ones_v, acc_sh.at[dstv.at[g]], add=True)
        return carry

    lax.fori_loop(0, CH_AGG, body, 0)
    plsc.subcore_barrier()
    pltpu.sync_copy(acc_sh.at[pl.ds(sid * ROWS_PER_SUB, ROWS_PER_SUB)],
                    out_hbm.at[cid, pl.ds(sid * ROWS_PER_SUB, ROWS_PER_SUB)])


def _agg_kernel(z_hbm, src_hbm, dst_hbm, zeros_hbm, out_hbm,
                acc_sh, rows0, rows1, srcv, dstv, sem0, sem1):
    cid = lax.axis_index("c")
    sid = lax.axis_index("s")
    wid = cid * NS + sid
    rows = (rows0, rows1)
    sems = (sem0, sem1)
    pltpu.sync_copy(zeros_hbm, acc_sh.at[pl.ds(sid * ROWS_PER_SUB, ROWS_PER_SUB)])
    pltpu.sync_copy(src_hbm.at[wid], srcv)
    pltpu.sync_copy(dst_hbm.at[wid], dstv)
    plsc.subcore_barrier()

    for b in range(NBUF):
        pltpu.async_copy(z_hbm.at[srcv.at[b]], rows[b], sems[b])

    def body(h, carry):
        for b in range(NBUF):
            g = h * NBUF + b
            pltpu.make_async_copy(z_hbm.at[srcv.at[0]], rows[b], sems[b]).wait()
            pltpu.sync_copy(rows[b], acc_sh.at[dstv.at[g]], add=True)

            @pl.when(g + NBUF < CH_AGG)
            def _():
                pltpu.async_copy(z_hbm.at[srcv.at[g + NBUF]], rows[b], sems[b])
        return carry

    lax.fori_loop(0, CH_AGG // NBUF, body, 0)
    plsc.subcore_barrier()
    pltpu.sync_copy(acc_sh.at[pl.ds(sid * ROWS_PER_SUB, ROWS_PER_SUB)],
                    out_hbm.at[cid, pl.ds(sid * ROWS_PER_SUB, ROWS_PER_SUB)])


def _make_deg():
    return pl.kernel(
        _deg_kernel,
        out_type=jax.ShapeDtypeStruct((NC, N_PAD, DEG_W), jnp.float32),
        mesh=_sc_mesh(),
        scratch_types=[
            pltpu.VMEM_SHARED((N_PAD, DEG_W), jnp.float32),
            pltpu.VMEM((CHUNK, DEG_W), jnp.float32),
            pltpu.VMEM((CH_AGG, CHUNK), jnp.int32),
            pltpu.SemaphoreType.DMA,
        ],
    )


def _make_agg():
    return pl.kernel(
        _agg_kernel,
        out_type=jax.ShapeDtypeStruct((NC, N_PAD, D_H), jnp.float32),
        mesh=_sc_mesh(),
        scratch_types=[
            pltpu.VMEM_SHARED((N_PAD, D_H), jnp.float32),
            pltpu.VMEM((CHUNK, D_H), jnp.float32),
            pltpu.VMEM((CHUNK, D_H), jnp.float32),
            pltpu.VMEM((CH_AGG, CHUNK), jnp.int32),
            pltpu.VMEM((CH_AGG, CHUNK), jnp.int32),
            pltpu.SemaphoreType.DMA,
            pltpu.SemaphoreType.DMA,
        ],
    )


def _dinv_of(degp_ref):
    d16 = lax.rsqrt(degp_ref[0] + degp_ref[1] + 1.0)
    return jnp.broadcast_to(d16[:, :1], (d16.shape[0], D_H))


def _k1_body(x_ref, w1_ref, degp_ref, z_ref):
    xw = jnp.dot(x_ref[...], w1_ref[...], preferred_element_type=jnp.float32)
    z_ref[...] = xw * _dinv_of(degp_ref)


def _k1(x_pad, W1, degp):
    return pl.pallas_call(
        _k1_body,
        grid=(NBLK,),
        in_specs=[
            pl.BlockSpec((BLK, D_IN), lambda i: (i, 0)),
            pl.BlockSpec((D_IN, D_H), lambda i: (0, 0)),
            pl.BlockSpec((NC, BLK, DEG_W), lambda i: (0, i, 0)),
        ],
        out_specs=pl.BlockSpec((BLK, D_H), lambda i: (i, 0)),
        out_shape=jax.ShapeDtypeStruct((N_PAD, D_H), jnp.float32),
    )(x_pad, W1, degp)


def _k2_body(s_ref, z1_ref, degp_ref, b1_ref, w2_ref, out_ref):
    dinv = _dinv_of(degp_ref)
    h = jnp.maximum(dinv * (s_ref[0] + s_ref[1] + z1_ref[...]) + b1_ref[...], 0.0)
    out_ref[...] = jnp.dot(h * dinv, w2_ref[...], preferred_element_type=jnp.float32)


def _k2(S1, Z1, degp, b1r, W2):
    return pl.pallas_call(
        _k2_body,
        grid=(NBLK,),
        in_specs=[
            pl.BlockSpec((NC, BLK, D_H), lambda i: (0, i, 0)),
            pl.BlockSpec((BLK, D_H), lambda i: (i, 0)),
            pl.BlockSpec((NC, BLK, DEG_W), lambda i: (0, i, 0)),
            pl.BlockSpec((1, D_H), lambda i: (0, 0)),
            pl.BlockSpec((D_H, D_H), lambda i: (0, 0)),
        ],
        out_specs=pl.BlockSpec((BLK, D_H), lambda i: (i, 0)),
        out_shape=jax.ShapeDtypeStruct((N_PAD, D_H), jnp.float32),
    )(S1, Z1, degp, b1r, W2)


def _k3_body(s_ref, z2_ref, degp_ref, b2_ref, batch_ref, wlin_ref, blin_ref,
             out_ref, sums_ref, cnt_ref):
    i = pl.program_id(0)

    @pl.when(i == 0)
    def _():
        sums_ref[...] = jnp.zeros_like(sums_ref)
        cnt_ref[...] = jnp.zeros_like(cnt_ref)

    dinv = _dinv_of(degp_ref)
    h2 = jnp.maximum(dinv * (s_ref[0] + s_ref[1] + z2_ref[...]) + b2_ref[...], 0.0)
    b = batch_ref[0]
    gids = lax.broadcasted_iota(jnp.int32, (G, BLK), 0)
    oh = (gids == b).astype(jnp.float32)
    sums_ref[...] += jnp.dot(oh, h2, preferred_element_type=jnp.float32)
    cnt_ref[...] += jnp.broadcast_to(
        jnp.sum(oh, axis=1, keepdims=True), (G, D_H))

    @pl.when(i == NBLK - 1)
    def _():
        pooled = sums_ref[...] / jnp.maximum(cnt_ref[...], 1.0)
        out_ref[...] = (
            jnp.dot(pooled, wlin_ref[...], preferred_element_type=jnp.float32)
            + blin_ref[...])


def _k3(S2, Z2, degp, b2r, batch3, wlin_pad, blin_pad):
    return pl.pallas_call(
        _k3_body,
        grid=(NBLK,),
        in_specs=[
            pl.BlockSpec((NC, BLK, D_H), lambda i: (0, i, 0)),
            pl.BlockSpec((BLK, D_H), lambda i: (i, 0)),
            pl.BlockSpec((NC, BLK, DEG_W), lambda i: (0, i, 0)),
            pl.BlockSpec((1, D_H), lambda i: (0, 0)),
            pl.BlockSpec((1, 1, BLK), lambda i: (i, 0, 0)),
            pl.BlockSpec((D_H, D_H), lambda i: (0, 0)),
            pl.BlockSpec((1, D_H), lambda i: (0, 0)),
        ],
        out_specs=pl.BlockSpec((G, D_H), lambda i: (0, 0)),
        out_shape=jax.ShapeDtypeStruct((G, D_H), jnp.float32),
        scratch_shapes=[
            pltpu.VMEM((G, D_H), jnp.float32),
            pltpu.VMEM((G, D_H), jnp.float32),
        ],
    )(S2, Z2, degp, b2r, batch3, wlin_pad, blin_pad)


@jax.jit
def kernel(x, edge_index, batch, W1, b1, W2, b2, Wlin, blin):
    src = edge_index[0]
    dst = edge_index[1]
    spread = N + (jnp.arange(E_PAD, dtype=jnp.int32) % (N_PAD - N))
    src_pad = spread.at[:E].set(src)
    dst_pad = spread.at[:E].set(dst)
    src3 = src_pad.reshape(NW, CH_AGG, CHUNK)
    dst3 = dst_pad.reshape(NW, CH_AGG, CHUNK)
    x_pad = jnp.zeros((N_PAD, D_IN), jnp.float32).at[:N, :].set(x)
    batch3 = jnp.full((N_PAD,), G, jnp.int32).at[:N].set(batch).reshape(NBLK, 1, BLK)
    ones_rows = jnp.ones((CHUNK, DEG_W), jnp.float32)
    zeros_deg = jnp.zeros((ROWS_PER_SUB, DEG_W), jnp.float32)
    zeros_rows = jnp.zeros((ROWS_PER_SUB, D_H), jnp.float32)
    b1r = b1.reshape(1, D_H)
    b2r = b2.reshape(1, D_H)
    wlin_pad = jnp.zeros((D_H, D_H), jnp.float32).at[:, :1].set(Wlin)
    blin_pad = jnp.zeros((1, D_H), jnp.float32).at[0, 0].set(blin[0])

    degp = _make_deg()(dst3, ones_rows, zeros_deg)
    Z1 = _k1(x_pad, W1, degp)
    S1 = _make_agg()(Z1, src3, dst3, zeros_rows)
    Z2 = _k2(S1, Z1, degp, b1r, W2)
    S2 = _make_agg()(Z2, src3, dst3, zeros_rows)
    outg = _k3(S2, Z2, degp, b2r, batch3, wlin_pad, blin_pad)
    return outg[:, 0].reshape(-1)
